# Initial kernel scaffold; baseline (speedup 1.0000x reference)
import jax
import jax.numpy as jnp
from jax.experimental import pallas as pl


def kernel(prv_varToFactor_messages, prv_factorToVar_messages, prv_factor_beliefs, factorToVar_edge_index, W1, b1, W2, b2):
    raise NotImplementedError("write your pallas kernel here")


# speedup vs baseline: 4.7962x; 4.7962x over previous
import functools

import jax
import jax.numpy as jnp
from jax import lax
from jax.experimental import pallas as pl
from jax.experimental.pallas import tpu as pltpu
from jax.experimental.pallas import tpu_sc as plsc

V = 100_000
F = 100_000
E = 1_600_000
S = 16
ALPHA = 0.5
SHIFT = 1e-19

NC = 2
NS = 16
NW = NC * NS
EPW = E // NW
IW = 125
BLK = 500
NROW = BLK // IW
NBLK = EPW // BLK
IROWS_PER_W = EPW // IW
WB = 6_248
WREM = V - NS * WB


def _zero_and_barrier(buf_v, acc_sh, s):
    @pl.loop(0, BLK)
    def _(e):
        buf_v[e] = jnp.zeros((S,), jnp.float32)

    base = pl.multiple_of(s * WB, 8)
    off = 0
    while off < WB:
        n = min(BLK, WB - off)
        pltpu.sync_copy(buf_v.at[pl.ds(0, n)], acc_sh.at[pl.ds(base + off, n)])
        off += n

    @pl.when(s == NS - 1)
    def _():
        pltpu.sync_copy(buf_v.at[pl.ds(0, WREM)], acc_sh.at[pl.ds(NS * WB, WREM)])

    plsc.subcore_barrier()


def _writeback(acc_sh, part_out, c, s, table_rows):
    base = pl.multiple_of(c * table_rows + s * WB, 8)
    pltpu.sync_copy(acc_sh.at[pl.ds(pl.multiple_of(s * WB, 8), WB)],
                    part_out.at[pl.ds(base, WB)])

    @pl.when(s == NS - 1)
    def _():
        tail = pl.multiple_of(c * table_rows + NS * WB, 8)
        pltpu.sync_copy(acc_sh.at[pl.ds(NS * WB, WREM)],
                        part_out.at[pl.ds(tail, WREM)])


def _pass1_body(fb_hbm, fidx_hbm, vidx_hbm, v2f_hbm, f2v_hbm,
                m_out, part_out,
                idxf_v, idxv_v, rows_v, v2f_v, f2v_v, acc_sh, sem):
    c = lax.axis_index("c")
    s = lax.axis_index("s")
    wid = c * NS + s

    _zero_and_barrier(rows_v, acc_sh, s)

    @pl.loop(0, NBLK)
    def _(blk):
        irow = wid * IROWS_PER_W + blk * NROW
        ebase = wid * EPW + blk * BLK
        pltpu.sync_copy(fidx_hbm.at[pl.ds(irow, NROW)], idxf_v)
        pltpu.sync_copy(vidx_hbm.at[pl.ds(irow, NROW)], idxv_v)
        gathers = [
            pltpu.async_copy(fb_hbm.at[idxf_v.at[j]],
                             rows_v.at[pl.ds(j * IW, IW)], sem)
            for j in range(NROW)
        ]
        pltpu.sync_copy(v2f_hbm.at[pl.ds(ebase, BLK)], v2f_v)
        pltpu.sync_copy(f2v_hbm.at[pl.ds(ebase, BLK)], f2v_v)
        for g in gathers:
            g.wait()

        @pl.loop(0, BLK, unroll=8)
        def _(e):
            rows_v[e] = (rows_v[e] - v2f_v[e] + f2v_v[e]) * ALPHA

        pltpu.sync_copy(rows_v, m_out.at[pl.ds(ebase, BLK)])
        scatters = [
            pltpu.async_copy(rows_v.at[pl.ds(j * IW, IW)],
                             acc_sh.at[idxv_v.at[j]], sem, add=True)
            for j in range(NROW)
        ]
        for sc in scatters:
            sc.wait()

    plsc.subcore_barrier()
    _writeback(acc_sh, part_out, c, s, V)


def _pass2_body(vb_hbm, vidx_hbm, fidx_hbm, m_hbm,
                v2f_out, part_out,
                idxv_v, idxf_v, rows_v, m_v, acc_sh, sem):
    c = lax.axis_index("c")
    s = lax.axis_index("s")
    wid = c * NS + s

    _zero_and_barrier(rows_v, acc_sh, s)

    @pl.loop(0, NBLK)
    def _(blk):
        irow = wid * IROWS_PER_W + blk * NROW
        ebase = wid * EPW + blk * BLK
        pltpu.sync_copy(vidx_hbm.at[pl.ds(irow, NROW)], idxv_v)
        pltpu.sync_copy(fidx_hbm.at[pl.ds(irow, NROW)], idxf_v)
        gathers = [
            pltpu.async_copy(vb_hbm.at[idxv_v.at[j]],
                             rows_v.at[pl.ds(j * IW, IW)], sem)
            for j in range(NROW)
        ]
        pltpu.sync_copy(m_hbm.at[pl.ds(ebase, BLK)], m_v)
        for g in gathers:
            g.wait()

        @pl.loop(0, BLK, unroll=8)
        def _(e):
            rows_v[e] = rows_v[e] - m_v[e]

        pltpu.sync_copy(rows_v, v2f_out.at[pl.ds(ebase, BLK)])
        scatters = [
            pltpu.async_copy(rows_v.at[pl.ds(j * IW, IW)],
                             acc_sh.at[idxf_v.at[j]], sem, add=True)
            for j in range(NROW)
        ]
        for sc in scatters:
            sc.wait()

    plsc.subcore_barrier()
    _writeback(acc_sh, part_out, c, s, F)


@functools.cache
def _build_sc_passes():
    mesh = plsc.VectorSubcoreMesh(core_axis_name="c", subcore_axis_name="s",
                                  num_cores=NC, num_subcores=NS)
    sc_params = pltpu.CompilerParams(use_tc_tiling_on_sc=False)
    pass1 = pl.kernel(
        _pass1_body,
        out_type=(
            jax.ShapeDtypeStruct((E, S), jnp.float32),
            jax.ShapeDtypeStruct((NC * V, S), jnp.float32),
        ),
        mesh=mesh,
        scratch_types=(
            pltpu.VMEM((NROW, IW), jnp.int32),
            pltpu.VMEM((NROW, IW), jnp.int32),
            pltpu.VMEM((BLK, S), jnp.float32),
            pltpu.VMEM((BLK, S), jnp.float32),
            pltpu.VMEM((BLK, S), jnp.float32),
            pltpu.VMEM_SHARED((V, S), jnp.float32),
            pltpu.SemaphoreType.DMA,
        ),
        compiler_params=sc_params,
    )
    pass2 = pl.kernel(
        _pass2_body,
        out_type=(
            jax.ShapeDtypeStruct((E, S), jnp.float32),
            jax.ShapeDtypeStruct((NC * F, S), jnp.float32),
        ),
        mesh=mesh,
        scratch_types=(
            pltpu.VMEM((NROW, IW), jnp.int32),
            pltpu.VMEM((NROW, IW), jnp.int32),
            pltpu.VMEM((BLK, S), jnp.float32),
            pltpu.VMEM((BLK, S), jnp.float32),
            pltpu.VMEM_SHARED((F, S), jnp.float32),
            pltpu.SemaphoreType.DMA,
        ),
        compiler_params=sc_params,
    )
    return pass1, pass2

TBLK = 2_000


def _norm_body(p_ref, o_ref):
    p = p_ref[...]
    vb = p[0] + p[1]
    mx = jnp.max(vb, axis=1, keepdims=True)
    lse = mx + jnp.log(jnp.sum(jnp.exp(vb - mx), axis=1, keepdims=True))
    o_ref[...] = vb - lse


_norm = pl.pallas_call(
    _norm_body,
    grid=(V // TBLK,),
    in_specs=[pl.BlockSpec((2, TBLK, S), lambda i: (0, i, 0))],
    out_specs=pl.BlockSpec((TBLK, S), lambda i: (i, 0)),
    out_shape=jax.ShapeDtypeStruct((V, S), jnp.float32),
)


def _mlp_body(p_ref, w1_ref, b1_ref, w2_ref, b2_ref, o_ref):
    p = p_ref[...]
    h = jnp.exp(p[0] + p[1])
    h = jax.lax.dot_general(h, w1_ref[...], (((1,), (1,)), ((), ())),
                            preferred_element_type=jnp.float32,
                            precision=jax.lax.Precision.HIGHEST) + b1_ref[...]
    h = jnp.maximum(h, 0.0)
    h = jax.lax.dot_general(h, w2_ref[...], (((1,), (1,)), ((), ())),
                            preferred_element_type=jnp.float32,
                            precision=jax.lax.Precision.HIGHEST) + b2_ref[...]
    h = SHIFT + jnp.maximum(h - SHIFT, 0.0)
    o_ref[...] = jnp.log(h)


_mlp = pl.pallas_call(
    _mlp_body,
    grid=(F // TBLK,),
    in_specs=[
        pl.BlockSpec((2, TBLK, S), lambda i: (0, i, 0)),
        pl.BlockSpec((S, S), lambda i: (0, 0)),
        pl.BlockSpec((1, S), lambda i: (0, 0)),
        pl.BlockSpec((S, S), lambda i: (0, 0)),
        pl.BlockSpec((1, S), lambda i: (0, 0)),
    ],
    out_specs=pl.BlockSpec((TBLK, S), lambda i: (i, 0)),
    out_shape=jax.ShapeDtypeStruct((F, S), jnp.float32),
)


def kernel(prv_varToFactor_messages, prv_factorToVar_messages, prv_factor_beliefs,
           factorToVar_edge_index, W1, b1, W2, b2):
    fidx2 = factorToVar_edge_index[0].reshape(E // IW, IW)
    vidx2 = factorToVar_edge_index[1].reshape(E // IW, IW)

    _pass1, _pass2 = _build_sc_passes()
    f2v_messages, vb_part = _pass1(
        prv_factor_beliefs, fidx2, vidx2,
        prv_varToFactor_messages, prv_factorToVar_messages)

    var_beliefs = _norm(vb_part.reshape(2, V, S))

    v2f_messages, fb_part = _pass2(var_beliefs, vidx2, fidx2, f2v_messages)

    factor_beliefs = _mlp(fb_part.reshape(2, F, S), W1, b1.reshape(1, S),
                          W2, b2.reshape(1, S))

    return v2f_messages, f2v_messages, var_beliefs, factor_beliefs
